# Initial kernel scaffold; baseline (speedup 1.0000x reference)
import jax
import jax.numpy as jnp
from jax.experimental import pallas as pl


def kernel(x, cls_token, Wp, bp, W_in, conv_w, conv_b, W_x, W_dt, b_dt, A_log, D_skip, W_out):
    raise NotImplementedError("write your pallas kernel here")


# speedup vs baseline: 897.4469x; 897.4469x over previous
import jax
import jax.numpy as jnp
from jax.experimental import pallas as pl

D_M = 256
D_STATE = 16
D_CONV = 4
D_INNER = 512
DT_RANK = 16


def _cls_mamba_kernel(cls_ref, W_in_ref, conv_tap_ref, conv_b_ref, W_x_ref,
                      W_dt_ref, b_dt_ref, D_skip_ref, W_out_ref, out_ref):
    c = cls_ref[:]
    xz = jnp.dot(c, W_in_ref[:], preferred_element_type=jnp.float32)
    xh = xz[:, :D_INNER]
    z = xz[:, D_INNER:]
    xc = xh * conv_tap_ref[:] + conv_b_ref[:]
    u = xc * jax.nn.sigmoid(xc)
    x_dbl = jnp.dot(u, W_x_ref[:], preferred_element_type=jnp.float32)
    dt = x_dbl[:, :DT_RANK]
    Bm = x_dbl[:, DT_RANK:DT_RANK + D_STATE]
    Cm = x_dbl[:, DT_RANK + D_STATE:]
    delta = jax.nn.softplus(
        jnp.dot(dt, W_dt_ref[:], preferred_element_type=jnp.float32)
        + b_dt_ref[:])
    bc = jnp.sum(Bm * Cm)
    y = delta * u * bc + u * D_skip_ref[:]
    y = y * (z * jax.nn.sigmoid(z))
    o = jnp.dot(y, W_out_ref[:], preferred_element_type=jnp.float32)
    out_ref[:] = jnp.broadcast_to(o, out_ref.shape)


def kernel(x, cls_token, Wp, bp, W_in, conv_w, conv_b, W_x, W_dt, b_dt,
           A_log, D_skip, W_out):
    batch = x.shape[0]
    cls2 = cls_token.reshape(1, D_M)
    conv_tap = conv_w[:, D_CONV - 1].reshape(1, D_INNER)
    return pl.pallas_call(
        _cls_mamba_kernel,
        out_shape=jax.ShapeDtypeStruct((batch, D_M), jnp.float32),
    )(cls2, W_in, conv_tap, conv_b.reshape(1, D_INNER), W_x, W_dt,
      b_dt.reshape(1, D_INNER), D_skip.reshape(1, D_INNER), W_out)
